# baseline (speedup 1.0000x reference)
import functools

import jax
import jax.numpy as jnp
from jax import lax
from jax.experimental import pallas as pl
from jax.experimental.pallas import tpu as pltpu
from jax.experimental.pallas import tpu_sc as plsc

D_MODEL = 64
SUB = 128
NSUB = 4
CHUNK = SUB * NSUB
NW = 32


@functools.partial(jax.jit, static_argnums=(2, 3))
def _gather_rows(idx2d, table, b_total, n_chunks):
    mesh = plsc.VectorSubcoreMesh(core_axis_name="c", subcore_axis_name="s")
    b_per_w = b_total // NW

    @functools.partial(
        pl.kernel,
        out_type=jax.ShapeDtypeStruct((b_total, D_MODEL), jnp.float32),
        mesh=mesh,
        scratch_types=[
            pltpu.VMEM((NSUB, SUB), jnp.int32),
            pltpu.VMEM((CHUNK, D_MODEL), jnp.float32),
            pltpu.SemaphoreType.DMA,
        ],
        compiler_params=pltpu.CompilerParams(use_tc_tiling_on_sc=False),
    )
    def k(idx_hbm, table_hbm, out_hbm, idx_v, rows_v, sem):
        wid = lax.axis_index("s") * 2 + lax.axis_index("c")
        idx_row0 = wid * (b_per_w // SUB)
        out_row0 = wid * b_per_w

        def body(ci, carry):
            pltpu.sync_copy(idx_hbm.at[pl.ds(idx_row0 + ci * NSUB, NSUB)],
                            idx_v)
            copies = []
            for j in range(NSUB):
                copies.append(pltpu.async_copy(
                    table_hbm.at[idx_v.at[j]],
                    rows_v.at[pl.ds(j * SUB, SUB)],
                    sem))
            for c in copies:
                c.wait()
            pltpu.sync_copy(rows_v,
                            out_hbm.at[pl.ds(out_row0 + ci * CHUNK, CHUNK)])
            return carry

        lax.fori_loop(0, n_chunks, body, 0)

    return k(idx2d, table)


def kernel(x, table):
    b, t = x.shape
    b_total = b * t
    idx2d = x.reshape(b_total // SUB, SUB)
    n_chunks = b_total // (NW * CHUNK)
    out = _gather_rows(idx2d, table, b_total, n_chunks)
    return out.reshape(b, t, D_MODEL)


# speedup vs baseline: 1.0691x; 1.0691x over previous
import functools

import jax
import jax.numpy as jnp
from jax import lax
from jax.experimental import pallas as pl
from jax.experimental.pallas import tpu as pltpu
from jax.experimental.pallas import tpu_sc as plsc

D_MODEL = 64
SUB = 128
NSUB = 5
CHUNK = SUB * NSUB
NW = 32


@functools.partial(jax.jit, static_argnums=(2, 3))
def _gather_rows(idx2d, table, b_total, n_chunks):
    mesh = plsc.VectorSubcoreMesh(core_axis_name="c", subcore_axis_name="s")
    b_per_w = b_total // NW

    @functools.partial(
        pl.kernel,
        out_type=jax.ShapeDtypeStruct((b_total, D_MODEL), jnp.float32),
        mesh=mesh,
        scratch_types=[
            pltpu.VMEM((NSUB, SUB), jnp.int32),
            pltpu.VMEM((NSUB, SUB), jnp.int32),
            pltpu.VMEM((CHUNK, D_MODEL), jnp.float32),
            pltpu.VMEM((CHUNK, D_MODEL), jnp.float32),
            pltpu.SemaphoreType.DMA,
            pltpu.SemaphoreType.DMA,
            pltpu.SemaphoreType.DMA,
            pltpu.SemaphoreType.DMA,
            pltpu.SemaphoreType.DMA,
            pltpu.SemaphoreType.DMA,
        ],
        compiler_params=pltpu.CompilerParams(use_tc_tiling_on_sc=False),
    )
    def k(idx_hbm, table_hbm, out_hbm, idx_v0, idx_v1, rows_v0, rows_v1,
          sem_i0, sem_i1, sem_g0, sem_g1, sem_o0, sem_o1):
        idx_bufs = [idx_v0, idx_v1]
        row_bufs = [rows_v0, rows_v1]
        sem_i = [sem_i0, sem_i1]
        sem_g = [sem_g0, sem_g1]
        sem_o = [sem_o0, sem_o1]

        wid = lax.axis_index("s") * 2 + lax.axis_index("c")
        idx_row0 = wid * (b_per_w // SUB)
        out_row0 = wid * b_per_w

        def idx_slice(ci):
            return idx_hbm.at[pl.ds(idx_row0 + ci * NSUB, NSUB)]

        def out_slice(ci):
            return out_hbm.at[pl.ds(out_row0 + ci * CHUNK, CHUNK)]

        for b in range(2):
            pltpu.async_copy(idx_slice(b), idx_bufs[b], sem_i[b])

        def body(g2, carry):
            g = g2 * 2
            for b in range(2):
                ci = g + b

                @pl.when(ci >= 2)
                def _wait_out():
                    pltpu.make_async_copy(
                        row_bufs[b], out_slice(ci), sem_o[b]).wait()

                pltpu.make_async_copy(
                    idx_slice(ci), idx_bufs[b], sem_i[b]).wait()

                copies = []
                for j in range(NSUB):
                    copies.append(pltpu.async_copy(
                        table_hbm.at[idx_bufs[b].at[j]],
                        row_bufs[b].at[pl.ds(j * SUB, SUB)],
                        sem_g[b]))
                for c in copies:
                    c.wait()

                @pl.when(ci + 2 < n_chunks)
                def _prefetch_idx():
                    pltpu.async_copy(idx_slice(ci + 2), idx_bufs[b],
                                     sem_i[b])

                pltpu.async_copy(row_bufs[b], out_slice(ci), sem_o[b])
            return carry

        lax.fori_loop(0, n_chunks // 2, body, 0)

        for b in range(2):
            pltpu.make_async_copy(
                row_bufs[b], out_slice(n_chunks - 2 + b), sem_o[b]).wait()

    return k(idx2d, table)


def kernel(x, table):
    b, t = x.shape
    b_total = b * t
    idx2d = x.reshape(b_total // SUB, SUB)
    n_chunks = b_total // (NW * CHUNK)
    out = _gather_rows(idx2d, table, b_total, n_chunks)
    return out.reshape(b, t, D_MODEL)
